# baseline (speedup 1.0000x reference)
import functools

import jax
import jax.numpy as jnp
from jax import lax
from jax.experimental import pallas as pl
from jax.experimental.pallas import tpu as pltpu
from jax.experimental.pallas import tpu_sc as plsc

_D = 64
_DPAD = 128
_SCALE = 8.0

_NC = 2
_NS = 16
_NW = _NC * _NS

_NBUF = 2
_HALF = 2
_UNROLL = 4


@functools.lru_cache(maxsize=None)
def _emb_fn(n_tok, seq):
    tok_per_w = n_tok // _NW
    chunk = tok_per_w // _HALF
    n_steps = seq * _HALF
    mesh = plsc.VectorSubcoreMesh(core_axis_name="c", subcore_axis_name="s")

    scratch = [pltpu.VMEM((seq, tok_per_w), jnp.int32)]
    scratch += [pltpu.VMEM((chunk, _DPAD), jnp.float32) for _ in range(_NBUF)]
    scratch += [pltpu.VMEM((chunk, _D), jnp.float32) for _ in range(_NBUF)]
    scratch += [pltpu.SemaphoreType.DMA for _ in range(2 * _NBUF + 1)]

    @functools.partial(
        pl.kernel,
        mesh=mesh,
        compiler_params=pltpu.CompilerParams(use_tc_tiling_on_sc=False),
        out_type=jax.ShapeDtypeStruct((seq, n_tok, _D), jnp.float32),
        scratch_types=scratch,
    )
    def emb(table_hbm, xt_hbm, out_hbm, xbuf, *rest):
        gbufs = rest[:_NBUF]
        sbufs = rest[_NBUF:2 * _NBUF]
        gsem = rest[2 * _NBUF:3 * _NBUF]
        osem = rest[3 * _NBUF:4 * _NBUF]
        xsem = rest[4 * _NBUF]

        wid = lax.axis_index("s") * _NC + lax.axis_index("c")
        tok0 = wid * tok_per_w
        pltpu.async_copy(
            xt_hbm.at[:, pl.ds(tok0, tok_per_w)], xbuf, xsem
        ).wait()

        def scale_buf(gbuf, sbuf):
            def body(i, carry):
                r0 = i * _UNROLL
                for dr in range(_UNROLL):
                    for k in range(_D // 16):
                        sl = pl.ds(k * 16, 16)
                        sbuf[r0 + dr, sl] = gbuf[r0 + dr, sl] * _SCALE
                return carry

            lax.fori_loop(0, chunk // _UNROLL, body, 0)

        def pair_body(g, carry):
            for b in range(_NBUF):
                step = g * _NBUF + b
                s = step // _HALF
                h = step % _HALF
                t_off = tok0 + h * chunk
                dst = out_hbm.at[s, pl.ds(t_off, chunk), :]

                @pl.when(g != 0)
                def _drain():
                    pltpu.make_async_copy(sbufs[b], dst, osem[b]).wait()

                pltpu.async_copy(
                    table_hbm.at[xbuf.at[s, pl.ds(h * chunk, chunk)]],
                    gbufs[b],
                    gsem[b],
                )
            for b in range(_NBUF):
                step = g * _NBUF + b
                s = step // _HALF
                h = step % _HALF
                t_off = tok0 + h * chunk
                dst = out_hbm.at[s, pl.ds(t_off, chunk), :]
                pltpu.make_async_copy(
                    table_hbm.at[xbuf.at[s, pl.ds(h * chunk, chunk)]],
                    gbufs[b],
                    gsem[b],
                ).wait()
                scale_buf(gbufs[b], sbufs[b])
                pltpu.async_copy(sbufs[b], dst, osem[b])
            return carry

        lax.fori_loop(0, n_steps // _NBUF, pair_body, 0)
        for b in range(_NBUF):
            step = n_steps - _NBUF + b
            s = step // _HALF
            h = step % _HALF
            dst = out_hbm.at[s, pl.ds(tok0 + h * chunk, chunk), :]
            pltpu.make_async_copy(sbufs[b], dst, osem[b]).wait()

    return emb


def kernel(x, embedding_weight):
    n_tok, seq = x.shape
    table_p = jnp.pad(embedding_weight, ((0, 0), (0, _DPAD - _D)))
    out_p = _emb_fn(n_tok, seq)(table_p, x.T)
    return out_p.transpose(1, 0, 2)
